# Initial kernel scaffold; baseline (speedup 1.0000x reference)
import jax
import jax.numpy as jnp
from jax.experimental import pallas as pl


def kernel(data, emb):
    raise NotImplementedError("write your pallas kernel here")


# speedup vs baseline: 1.1116x; 1.1116x over previous
import functools

import jax
import jax.numpy as jnp
from jax import lax
from jax.experimental import pallas as pl
from jax.experimental.pallas import tpu as pltpu
from jax.experimental.pallas import tpu_sc as plsc

N_ROWS = 16384
N_COLS = 50
D = 32
B = N_ROWS * N_COLS

NC = 2
NS = 16
NW = NC * NS
PER_W = B // NW
CH = 128
NCH = PER_W // CH
NBUF = 8
IDX_ROWS = B // CH

_mesh = plsc.VectorSubcoreMesh(core_axis_name="c", subcore_axis_name="s")


@functools.partial(
    pl.kernel,
    out_type=jax.ShapeDtypeStruct((B, D), jnp.float32),
    mesh=_mesh,
    scratch_types=[
        pltpu.VMEM((NCH, CH), jnp.int32),
        pltpu.VMEM((NBUF, CH, D), jnp.float32),
        pltpu.SemaphoreType.DMA((NBUF,)),
        pltpu.SemaphoreType.DMA((NBUF,)),
    ],
    compiler_params=pltpu.CompilerParams(use_tc_tiling_on_sc=False),
)
def _sc_gather(idx_hbm, emb_hbm, out_hbm, idx_v, bufs, gsem, wsem):
    wid = lax.axis_index("s") * NC + lax.axis_index("c")
    row0 = wid * NCH
    base = wid * PER_W

    pltpu.sync_copy(idx_hbm.at[pl.ds(row0, NCH)], idx_v)

    for b in range(NBUF):
        pltpu.async_copy(emb_hbm.at[idx_v.at[b]], bufs.at[b], gsem.at[b])

    def group(gi, carry):
        g = gi * NBUF
        writes = []
        for b in range(NBUF):
            j = g + b
            pltpu.make_async_copy(
                emb_hbm.at[idx_v.at[j]], bufs.at[b], gsem.at[b]).wait()
            writes.append(pltpu.async_copy(
                bufs.at[b], out_hbm.at[pl.ds(base + j * CH, CH)], wsem.at[b]))
        for b in range(NBUF):
            jn = g + NBUF + b
            writes[b].wait()
            pltpu.async_copy(
                emb_hbm.at[idx_v.at[jn]], bufs.at[b], gsem.at[b])
        return carry

    lax.fori_loop(0, NCH // NBUF - 1, group, 0)

    g = NCH - NBUF
    for b in range(NBUF):
        j = g + b
        pltpu.make_async_copy(
            emb_hbm.at[idx_v.at[j]], bufs.at[b], gsem.at[b]).wait()
        pltpu.async_copy(
            bufs.at[b], out_hbm.at[pl.ds(base + j * CH, CH)], wsem.at[b]).wait()


def kernel(data, emb):
    idx = data.reshape(-1).astype(jnp.int32).reshape(IDX_ROWS, CH)
    out = _sc_gather(idx, emb)
    return out.reshape(N_ROWS, N_COLS, D)


# speedup vs baseline: 1.4439x; 1.2989x over previous
import functools

import jax
import jax.numpy as jnp
from jax import lax
from jax.experimental import pallas as pl
from jax.experimental.pallas import tpu as pltpu
from jax.experimental.pallas import tpu_sc as plsc

N_I = 16384
N_J = 50
D = 32

NC = 2
NS = 16
NW = NC * NS
I_PER_W = N_I // NW
NT = I_PER_W // 128
NBUF = NT

_mesh = plsc.VectorSubcoreMesh(core_axis_name="c", subcore_axis_name="s")


@functools.partial(
    pl.kernel,
    out_type=jax.ShapeDtypeStruct((N_J, D // 8, N_I // 128, 8, 128), jnp.float32),
    mesh=_mesh,
    scratch_types=[
        pltpu.VMEM((N_J, I_PER_W), jnp.int32),
        pltpu.VMEM((NBUF, 128, D), jnp.float32),
        pltpu.VMEM((NBUF, D // 8, 8, 128), jnp.float32),
        pltpu.SemaphoreType.DMA((NBUF,)),
        pltpu.SemaphoreType.DMA((NBUF,)),
    ],
    compiler_params=pltpu.CompilerParams(
        use_tc_tiling_on_sc=False, needs_layout_passes=False),
)
def _sc_gather(dataT_hbm, emb_hbm, out5_hbm, idxT_v, gbufs, lbufs, gsem, wsem):
    wid = lax.axis_index("s") * NC + lax.axis_index("c")
    i0 = wid * I_PER_W

    pltpu.sync_copy(dataT_hbm.at[:, pl.ds(i0, I_PER_W)], idxT_v)

    iota = lax.iota(jnp.int32, 16)

    def jbody(j, carry):
        for t in range(NT):
            @pl.when(j > 0)
            def _():
                for tc in range(D // 8):
                    pltpu.make_async_copy(
                        lbufs.at[t, tc],
                        out5_hbm.at[j, tc, wid * NT + t],
                        wsem.at[t]).wait()
            pltpu.async_copy(
                emb_hbm.at[idxT_v.at[j, pl.ds(t * 128, 128)]],
                gbufs.at[t], gsem.at[t])
        for t in range(NT):
            pltpu.make_async_copy(
                emb_hbm.at[idxT_v.at[j, pl.ds(t * 128, 128)]],
                gbufs.at[t], gsem.at[t]).wait()
            for tc in range(D // 8):
                for s in range(8):
                    c = tc * 8 + s
                    cvec = jnp.full((16,), c, jnp.int32)
                    for l0 in range(0, 128, 16):
                        v = plsc.load_gather(gbufs.at[t], [l0 + iota, cvec])
                        lbufs[t, tc, s, pl.ds(l0, 16)] = v
            for tc in range(D // 8):
                pltpu.async_copy(
                    lbufs.at[t, tc],
                    out5_hbm.at[j, tc, wid * NT + t],
                    wsem.at[t])
        return carry

    lax.fori_loop(0, N_J, jbody, 0)

    for t in range(NT):
        for tc in range(D // 8):
            pltpu.make_async_copy(
                lbufs.at[t, tc],
                out5_hbm.at[N_J - 1, tc, wid * NT + t],
                wsem.at[t]).wait()


def kernel(data, emb):
    out5 = _sc_gather(data.T.astype(jnp.int32), emb)
    return out5.transpose(2, 4, 0, 1, 3).reshape(N_I, N_J, D)


# speedup vs baseline: 1.7718x; 1.2271x over previous
import functools

import jax
import jax.numpy as jnp
from jax import lax
from jax.experimental import pallas as pl
from jax.experimental.pallas import tpu as pltpu
from jax.experimental.pallas import tpu_sc as plsc

N_I = 16384
N_J = 50
D = 32

NC = 2
NS = 16
NW = NC * NS
I_PER_W = N_I // NW
NT = I_PER_W // 128
NBUF = 2 * NT

_mesh = plsc.VectorSubcoreMesh(core_axis_name="c", subcore_axis_name="s")


@functools.partial(
    pl.kernel,
    out_type=jax.ShapeDtypeStruct((N_J, D // 8, N_I // 128, 8, 128), jnp.float32),
    mesh=_mesh,
    scratch_types=[
        pltpu.VMEM((N_J, I_PER_W), jnp.int32),
        pltpu.VMEM((NBUF, 128, D), jnp.float32),
        pltpu.VMEM((NBUF, D // 8, 8, 128), jnp.float32),
        pltpu.SemaphoreType.DMA((NBUF,)),
        pltpu.SemaphoreType.DMA((NBUF,)),
    ],
    compiler_params=pltpu.CompilerParams(
        use_tc_tiling_on_sc=False, needs_layout_passes=False),
)
def _sc_gather(dataT_hbm, emb_hbm, out5_hbm, idxT_v, gbufs, lbufs, gsem, wsem):
    wid = lax.axis_index("s") * NC + lax.axis_index("c")
    i0 = wid * I_PER_W

    pltpu.sync_copy(dataT_hbm.at[:, pl.ds(i0, I_PER_W)], idxT_v)

    iota = lax.iota(jnp.int32, 16)
    rowvecs = [l0 + iota for l0 in range(0, 128, 16)]

    def fire_gathers(j, base):
        for t in range(NT):
            pltpu.async_copy(
                emb_hbm.at[idxT_v.at[j, pl.ds(t * 128, 128)]],
                gbufs.at[base + t], gsem.at[base + t])

    def drain_gathers(j, base):
        for t in range(NT):
            pltpu.make_async_copy(
                emb_hbm.at[idxT_v.at[j, pl.ds(t * 128, 128)]],
                gbufs.at[base + t], gsem.at[base + t]).wait()

    def drain_writes(j, base):
        for t in range(NT):
            for tc in range(D // 8):
                pltpu.make_async_copy(
                    lbufs.at[base + t, tc],
                    out5_hbm.at[j, tc, wid * NT + t],
                    wsem.at[base + t]).wait()

    def transpose_and_write(j, base):
        for t in range(NT):
            for c in range(D):
                cvec = jnp.full((16,), c, jnp.int32)
                vs = [plsc.load_gather(gbufs.at[base + t], [rv, cvec])
                      for rv in rowvecs]
                for k, v in enumerate(vs):
                    lbufs[base + t, c // 8, c % 8, pl.ds(k * 16, 16)] = v
            for tc in range(D // 8):
                pltpu.async_copy(
                    lbufs.at[base + t, tc],
                    out5_hbm.at[j, tc, wid * NT + t],
                    wsem.at[base + t])

    fire_gathers(0, 0)

    def gbody(g, carry):
        j0 = 2 * g
        fire_gathers(j0 + 1, NT)

        drain_gathers(j0, 0)

        @pl.when(g > 0)
        def _():
            drain_writes(j0 - 2, 0)
        transpose_and_write(j0, 0)

        @pl.when(g < (N_J // 2 - 1))
        def _():
            fire_gathers(j0 + 2, 0)

        drain_gathers(j0 + 1, NT)

        @pl.when(g > 0)
        def _():
            drain_writes(j0 - 1, NT)
        transpose_and_write(j0 + 1, NT)
        return carry

    lax.fori_loop(0, N_J // 2, gbody, 0)

    drain_writes(N_J - 2, 0)
    drain_writes(N_J - 1, NT)


def kernel(data, emb):
    out5 = _sc_gather(data.T.astype(jnp.int32), emb)
    return out5.transpose(2, 4, 0, 1, 3).reshape(N_I, N_J, D)


# speedup vs baseline: 1.8191x; 1.0267x over previous
import functools

import jax
import jax.numpy as jnp
from jax import lax
from jax.experimental import pallas as pl
from jax.experimental.pallas import tpu as pltpu
from jax.experimental.pallas import tpu_sc as plsc

N_I = 16384
N_J = 50
D = 32

NC = 2
NS = 16
NW = NC * NS
I_PER_W = N_I // NW
NT = I_PER_W // 128
NBUF = 2 * NT

_mesh = plsc.VectorSubcoreMesh(core_axis_name="c", subcore_axis_name="s")


@functools.partial(
    pl.kernel,
    out_type=jax.ShapeDtypeStruct((N_J, D // 8, N_I // 128, 8 * 128), jnp.float32),
    mesh=_mesh,
    scratch_types=[
        pltpu.VMEM((N_J, I_PER_W), jnp.int32),
        pltpu.VMEM((NBUF, 128, D), jnp.float32),
        pltpu.VMEM((NBUF, D * 128), jnp.float32),
        pltpu.SemaphoreType.DMA((NBUF,)),
        pltpu.SemaphoreType.DMA((NBUF,)),
    ],
    compiler_params=pltpu.CompilerParams(
        use_tc_tiling_on_sc=False, needs_layout_passes=False,
        disable_bounds_checks=True),
)
def _sc_gather(dataT_hbm, emb_hbm, out5_hbm, idxT_v, gbufs, lbufs, gsem, wsem):
    wid = lax.axis_index("s") * NC + lax.axis_index("c")
    i0 = wid * I_PER_W

    pltpu.sync_copy(dataT_hbm.at[:, pl.ds(i0, I_PER_W)], idxT_v)

    iota = lax.iota(jnp.int32, 16)
    basevecs = [(16 * h + iota) * 128 for h in range(2)]

    def fire_gathers(j, base):
        for t in range(NT):
            pltpu.async_copy(
                emb_hbm.at[idxT_v.at[j, pl.ds(t * 128, 128)]],
                gbufs.at[base + t], gsem.at[base + t])

    def drain_gathers(j, base):
        for t in range(NT):
            pltpu.make_async_copy(
                emb_hbm.at[idxT_v.at[j, pl.ds(t * 128, 128)]],
                gbufs.at[base + t], gsem.at[base + t]).wait()

    def drain_writes(j, base):
        for t in range(NT):
            for tc in range(D // 8):
                pltpu.make_async_copy(
                    lbufs.at[base + t, pl.ds(tc * 1024, 1024)],
                    out5_hbm.at[j, tc, wid * NT + t],
                    wsem.at[base + t]).wait()

    def transpose_and_write(j, base):
        for t in range(NT):
            for l in range(128):
                for h in range(2):
                    v = gbufs[base + t, l, pl.ds(16 * h, 16)]
                    plsc.store_scatter(
                        lbufs.at[base + t], [basevecs[h] + l], v)
            for tc in range(D // 8):
                pltpu.async_copy(
                    lbufs.at[base + t, pl.ds(tc * 1024, 1024)],
                    out5_hbm.at[j, tc, wid * NT + t],
                    wsem.at[base + t])

    fire_gathers(0, 0)

    def gbody(g, carry):
        j0 = 2 * g
        fire_gathers(j0 + 1, NT)

        drain_gathers(j0, 0)

        @pl.when(g > 0)
        def _():
            drain_writes(j0 - 2, 0)
        transpose_and_write(j0, 0)

        @pl.when(g < (N_J // 2 - 1))
        def _():
            fire_gathers(j0 + 2, 0)

        drain_gathers(j0 + 1, NT)

        @pl.when(g > 0)
        def _():
            drain_writes(j0 - 1, NT)
        transpose_and_write(j0 + 1, NT)
        return carry

    lax.fori_loop(0, N_J // 2, gbody, 0)

    drain_writes(N_J - 2, 0)
    drain_writes(N_J - 1, NT)


def kernel(data, emb):
    out5 = _sc_gather(data.T.astype(jnp.int32), emb)
    out5 = out5.reshape(N_J, D // 8, N_I // 128, 8, 128)
    return out5.transpose(2, 4, 0, 1, 3).reshape(N_I, N_J, D)


# speedup vs baseline: 2.5817x; 1.4193x over previous
import functools

import jax
import jax.numpy as jnp
from jax import lax
from jax.experimental import pallas as pl
from jax.experimental.pallas import tpu as pltpu
from jax.experimental.pallas import tpu_sc as plsc

N_I = 16384
N_J = 50
D = 32

NC = 2
NS = 16
NW = NC * NS
I_PER_W = N_I // NW
NT = I_PER_W // 128
NBUF = 2 * NT

_mesh = plsc.VectorSubcoreMesh(core_axis_name="c", subcore_axis_name="s")


@functools.partial(
    pl.kernel,
    out_type=jax.ShapeDtypeStruct((N_J, D // 8, N_I // 128, 8 * 128), jnp.float32),
    mesh=_mesh,
    scratch_types=[
        pltpu.VMEM((N_J, I_PER_W), jnp.int32),
        pltpu.VMEM((NBUF, 128, D), jnp.float32),
        pltpu.VMEM((NBUF, D * 128), jnp.float32),
        pltpu.SemaphoreType.DMA((NBUF,)),
        pltpu.SemaphoreType.DMA((NBUF,)),
    ],
    compiler_params=pltpu.CompilerParams(
        use_tc_tiling_on_sc=False, needs_layout_passes=False,
        disable_bounds_checks=True),
)
def _sc_gather(dataT_hbm, emb_hbm, out5_hbm, idxT_v, gbufs, lbufs, gsem, wsem):
    wid = lax.axis_index("s") * NC + lax.axis_index("c")
    i0 = wid * I_PER_W

    pltpu.sync_copy(dataT_hbm.at[:, pl.ds(i0, I_PER_W)], idxT_v)

    iota = lax.iota(jnp.int32, 16)

    def fire_gathers(j, base):
        for t in range(NT):
            pltpu.async_copy(
                emb_hbm.at[idxT_v.at[j, pl.ds(t * 128, 128)]],
                gbufs.at[base + t], gsem.at[base + t])

    def drain_gathers(j, base):
        for t in range(NT):
            pltpu.make_async_copy(
                emb_hbm.at[idxT_v.at[j, pl.ds(t * 128, 128)]],
                gbufs.at[base + t], gsem.at[base + t]).wait()

    def drain_writes(j, base):
        for t in range(NT):
            for tc in range(D // 8):
                pltpu.make_async_copy(
                    lbufs.at[base + t, pl.ds(tc * 1024, 1024)],
                    out5_hbm.at[j, tc, wid * NT + t],
                    wsem.at[base + t]).wait()

    def transpose_and_write(j, base):
        for t in range(NT):
            def kbody(k, carry):
                dk = (iota + k) & 15
                sk = dk * 128 + iota
                for L in range(8):
                    rL = 16 * L + iota
                    for h in range(2):
                        v = plsc.load_gather(
                            gbufs.at[base + t], [rL, dk + 16 * h])
                        plsc.store_scatter(
                            lbufs.at[base + t],
                            [sk + (2048 * h + 16 * L)], v)
                return carry
            lax.fori_loop(0, 16, kbody, 0)
            for tc in range(D // 8):
                pltpu.async_copy(
                    lbufs.at[base + t, pl.ds(tc * 1024, 1024)],
                    out5_hbm.at[j, tc, wid * NT + t],
                    wsem.at[base + t])

    fire_gathers(0, 0)

    def gbody(g, carry):
        j0 = 2 * g
        fire_gathers(j0 + 1, NT)

        drain_gathers(j0, 0)

        @pl.when(g > 0)
        def _():
            drain_writes(j0 - 2, 0)
        transpose_and_write(j0, 0)

        @pl.when(g < (N_J // 2 - 1))
        def _():
            fire_gathers(j0 + 2, 0)

        drain_gathers(j0 + 1, NT)

        @pl.when(g > 0)
        def _():
            drain_writes(j0 - 1, NT)
        transpose_and_write(j0 + 1, NT)
        return carry

    lax.fori_loop(0, N_J // 2, gbody, 0)

    drain_writes(N_J - 2, 0)
    drain_writes(N_J - 1, NT)


def kernel(data, emb):
    out5 = _sc_gather(data.T.astype(jnp.int32), emb)
    out5 = out5.reshape(N_J, D // 8, N_I // 128, 8, 128)
    return out5.transpose(2, 4, 0, 1, 3).reshape(N_I, N_J, D)


# speedup vs baseline: 3.7510x; 1.4529x over previous
import functools

import jax
import jax.numpy as jnp
from jax import lax
from jax.experimental import pallas as pl
from jax.experimental.pallas import tpu as pltpu
from jax.experimental.pallas import tpu_sc as plsc

N_I = 16384
N_J = 50
D = 32

NC = 2
NS = 16
NW = NC * NS
I_PER_W = N_I // NW
NT = I_PER_W // 128
NBUF = 2 * NT

_mesh = plsc.VectorSubcoreMesh(core_axis_name="c", subcore_axis_name="s")

N_EMB_ROWS = 1000000
N_UNITS = N_EMB_ROWS // 128
N_FULL = N_UNITS - N_UNITS % NW
NBA = 2


@functools.partial(
    pl.kernel,
    out_type=jax.ShapeDtypeStruct((N_EMB_ROWS * D,), jnp.float32),
    mesh=_mesh,
    scratch_types=[
        pltpu.VMEM((D, 128), jnp.float32),
        pltpu.VMEM((D, 128), jnp.float32),
        pltpu.VMEM((D * 128,), jnp.float32),
        pltpu.VMEM((D * 128,), jnp.float32),
        pltpu.VMEM((D, 64), jnp.float32),
        pltpu.VMEM((D * 64,), jnp.float32),
        pltpu.SemaphoreType.DMA((NBA,)),
        pltpu.SemaphoreType.DMA((NBA,)),
        pltpu.SemaphoreType.DMA,
    ],
    compiler_params=pltpu.CompilerParams(
        use_tc_tiling_on_sc=True, needs_layout_passes=False,
        disable_bounds_checks=True),
)
def _sc_repack(embT_hbm, wide_hbm, ibuf0, ibuf1, obuf0, obuf1,
               tibuf, tobuf, gsem, wsem, tsem):
    ibufs = [ibuf0, ibuf1]
    obufs = [obuf0, obuf1]
    wid = lax.axis_index("s") * NC + lax.axis_index("c")
    iota = lax.iota(jnp.int32, 16)

    def fire_in(u, b):
        pltpu.async_copy(
            embT_hbm.at[:, pl.ds(u * 128, 128)], ibufs[b], gsem.at[b])

    def drain_in(u, b):
        pltpu.make_async_copy(
            embT_hbm.at[:, pl.ds(u * 128, 128)], ibufs[b], gsem.at[b]).wait()

    def transpose_unit(src, dst, nl):
        def kbody(k, carry):
            dk = (iota + k) & 15
            sk = (iota // 4) * 128 + (iota % 4) * 32 + dk
            for L in range(nl):
                cols = 16 * L + iota
                for h in range(2):
                    v = plsc.load_gather(src, [dk + 16 * h, cols])
                    plsc.store_scatter(dst, [sk + (512 * L + 16 * h)], v)
            return carry
        lax.fori_loop(0, 16, kbody, 0)

    def fire_out(u, b):
        pltpu.async_copy(
            obufs[b], wide_hbm.at[pl.ds(u * 4096, 4096)], wsem.at[b])

    def drain_out(u, b):
        pltpu.make_async_copy(
            obufs[b], wide_hbm.at[pl.ds(u * 4096, 4096)], wsem.at[b]).wait()

    fire_in(wid, 0)

    def gbody(g, carry):
        u0 = wid + NW * 2 * g
        u1 = u0 + NW

        @pl.when(u1 < N_FULL)
        def _():
            fire_in(u1, 1)
        drain_in(u0, 0)

        @pl.when(g > 0)
        def _():
            drain_out(u0 - 2 * NW, 0)
        transpose_unit(ibufs[0], obufs[0], 8)
        fire_out(u0, 0)

        @pl.when(u1 + NW < N_FULL)
        def _():
            fire_in(u1 + NW, 0)

        @pl.when(u1 < N_FULL)
        def _():
            drain_in(u1, 1)

            @pl.when(g > 0)
            def _():
                drain_out(u1 - 2 * NW, 1)
            transpose_unit(ibufs[1], obufs[1], 8)
            fire_out(u1, 1)
        return carry

    n_g = N_FULL // (2 * NW)
    lax.fori_loop(0, n_g, gbody, 0)
    drain_out(N_FULL - 2 * NW + wid, 0)
    drain_out(N_FULL - NW + wid, 1)

    @pl.when(wid < 4)
    def _():
        u = N_FULL + wid
        pltpu.sync_copy(embT_hbm.at[:, pl.ds(u * 128, 128)], ibufs[0])
        transpose_unit(ibufs[0], obufs[0], 8)
        pltpu.async_copy(
            obufs[0], wide_hbm.at[pl.ds(u * 4096, 4096)], wsem.at[0]).wait()

    @pl.when(wid == 4)
    def _():
        c0 = N_UNITS * 128
        pltpu.sync_copy(embT_hbm.at[:, pl.ds(c0, 64)], tibuf)
        transpose_unit(tibuf, tobuf, 4)
        pltpu.async_copy(
            tobuf, wide_hbm.at[pl.ds(c0 * D, 64 * D)], tsem).wait()


@functools.partial(
    pl.kernel,
    out_type=jax.ShapeDtypeStruct((N_J, D // 8, N_I // 128, 8 * 128), jnp.float32),
    mesh=_mesh,
    scratch_types=[
        pltpu.VMEM((N_J, I_PER_W), jnp.int32),
        pltpu.VMEM((NBUF, 128, D), jnp.float32),
        pltpu.VMEM((NBUF, D * 128), jnp.float32),
        pltpu.SemaphoreType.DMA((NBUF,)),
        pltpu.SemaphoreType.DMA((NBUF,)),
    ],
    compiler_params=pltpu.CompilerParams(
        use_tc_tiling_on_sc=False, needs_layout_passes=False,
        disable_bounds_checks=True),
)
def _sc_gather(dataT_hbm, emb_hbm, out5_hbm, idxT_v, gbufs, lbufs, gsem, wsem):
    wid = lax.axis_index("s") * NC + lax.axis_index("c")
    i0 = wid * I_PER_W

    pltpu.sync_copy(dataT_hbm.at[:, pl.ds(i0, I_PER_W)], idxT_v)

    iota = lax.iota(jnp.int32, 16)

    def fire_gathers(j, base):
        for t in range(NT):
            pltpu.async_copy(
                emb_hbm.at[idxT_v.at[j, pl.ds(t * 128, 128)]],
                gbufs.at[base + t], gsem.at[base + t])

    def drain_gathers(j, base):
        for t in range(NT):
            pltpu.make_async_copy(
                emb_hbm.at[idxT_v.at[j, pl.ds(t * 128, 128)]],
                gbufs.at[base + t], gsem.at[base + t]).wait()

    def drain_writes(j, base):
        for t in range(NT):
            for tc in range(D // 8):
                pltpu.make_async_copy(
                    lbufs.at[base + t, pl.ds(tc * 1024, 1024)],
                    out5_hbm.at[j, tc, wid * NT + t],
                    wsem.at[base + t]).wait()

    def transpose_and_write(j, base):
        for t in range(NT):
            def kbody(k, carry):
                dk = (iota + k) & 15
                sk = dk * 128 + iota
                for L in range(8):
                    rL = 16 * L + iota
                    for h in range(2):
                        v = plsc.load_gather(
                            gbufs.at[base + t], [rL, dk + 16 * h])
                        plsc.store_scatter(
                            lbufs.at[base + t],
                            [sk + (2048 * h + 16 * L)], v)
                return carry
            lax.fori_loop(0, 16, kbody, 0)
            for tc in range(D // 8):
                pltpu.async_copy(
                    lbufs.at[base + t, pl.ds(tc * 1024, 1024)],
                    out5_hbm.at[j, tc, wid * NT + t],
                    wsem.at[base + t])

    fire_gathers(0, 0)

    def gbody(g, carry):
        j0 = 2 * g
        fire_gathers(j0 + 1, NT)

        drain_gathers(j0, 0)

        @pl.when(g > 0)
        def _():
            drain_writes(j0 - 2, 0)
        transpose_and_write(j0, 0)

        @pl.when(g < (N_J // 2 - 1))
        def _():
            fire_gathers(j0 + 2, 0)

        drain_gathers(j0 + 1, NT)

        @pl.when(g > 0)
        def _():
            drain_writes(j0 - 1, NT)
        transpose_and_write(j0 + 1, NT)
        return carry

    lax.fori_loop(0, N_J // 2, gbody, 0)

    drain_writes(N_J - 2, 0)
    drain_writes(N_J - 1, NT)


def kernel(data, emb):
    wide = _sc_repack(emb.T)
    out5 = _sc_gather(data.T.astype(jnp.int32), wide.reshape(N_EMB_ROWS, D))
    out5 = out5.reshape(N_J, D // 8, N_I // 128, 8, 128)
    return out5.transpose(2, 4, 0, 1, 3).reshape(N_I, N_J, D)
